# Initial kernel scaffold; baseline (speedup 1.0000x reference)
import jax
import jax.numpy as jnp
from jax.experimental import pallas as pl


def kernel(x, edge_index, edge_weight, W1_rel, b1_rel, W1_root, b1_root, W2_rel, b2_rel, W2_root, b2_root):
    raise NotImplementedError("write your pallas kernel here")


# speedup vs baseline: 6.5152x; 6.5152x over previous
import functools

import jax
import jax.numpy as jnp
from jax import lax
from jax.experimental import pallas as pl
from jax.experimental.pallas import tpu as pltpu
from jax.experimental.pallas import tpu_sc as plsc

N = 10000
E = 320000
D = 128

NC = 2
NS = 16
NW = NC * NS
EW = E // NW
C = 80
NCH = EW // C
NP = 10240
RPT = NP // NS
ZR = 64
CB = 25
NSB = NCH // CB
LANES = 16


def _sc_scatter_body(with_weight, y_hbm, src_hbm, dst_hbm, w_hbm, out_hbm,
                     src_v, dst_v, w_v, rows_v, stage_v, acc_sh, sem):
    cid = lax.axis_index("c")
    sid = lax.axis_index("s")
    wid = sid * NC + cid

    def zero_body(i, _):
        stage_v[i // 8, pl.ds((i % 8) * LANES, LANES)] = jnp.zeros(
            (LANES,), jnp.float32)
        return 0
    lax.fori_loop(0, ZR * 8, zero_body, 0)
    for z in range(RPT // ZR):
        pltpu.sync_copy(stage_v, acc_sh.at[pl.ds(sid * RPT + z * ZR, ZR)])
    plsc.subcore_barrier()

    def sblock_body(sb, _):
        pltpu.sync_copy(src_hbm.at[wid, sb], src_v)
        pltpu.sync_copy(dst_hbm.at[wid, sb], dst_v)
        if with_weight:
            pltpu.sync_copy(w_hbm.at[wid, sb], w_v)

        def chunk_body(j, _):
            pltpu.async_copy(y_hbm.at[src_v.at[j]], rows_v, sem).wait()
            if with_weight:
                def group_body(g, _):
                    wv = w_v[pl.ds(j * C + g * LANES, LANES)]
                    for el in range(LANES):
                        e = g * LANES + el
                        wb = jnp.full((LANES,), wv[el], jnp.float32)
                        for k in range(D // LANES):
                            sl = pl.ds(k * LANES, LANES)
                            rows_v[e, sl] = rows_v[e, sl] * wb
                    return 0
                lax.fori_loop(0, C // LANES, group_body, 0)
            pltpu.sync_copy(rows_v, acc_sh.at[dst_v.at[j]], add=True)
            return 0
        lax.fori_loop(0, CB, chunk_body, 0)
        return 0
    lax.fori_loop(0, NSB, sblock_body, 0)
    plsc.subcore_barrier()

    for z in range(RPT // ZR):
        r0 = sid * RPT + z * ZR
        pltpu.sync_copy(acc_sh.at[pl.ds(r0, ZR)], stage_v)
        pltpu.sync_copy(stage_v, out_hbm.at[cid, pl.ds(r0, ZR)])


def _make_sc_scatter(with_weight):
    mesh = plsc.VectorSubcoreMesh(core_axis_name="c", subcore_axis_name="s",
                                  num_cores=NC, num_subcores=NS)
    scratch = [
        pltpu.VMEM((CB, C), jnp.int32),
        pltpu.VMEM((CB, C), jnp.int32),
        pltpu.VMEM((CB * C,), jnp.float32),
        pltpu.VMEM((C, D), jnp.float32),
        pltpu.VMEM((ZR, D), jnp.float32),
        pltpu.VMEM_SHARED((NP, D), jnp.float32),
        pltpu.SemaphoreType.DMA,
    ]
    return pl.kernel(
        functools.partial(_sc_scatter_body, with_weight),
        out_type=jax.ShapeDtypeStruct((NC, NP, D), jnp.float32),
        mesh=mesh,
        scratch_types=scratch,
        name="sc_scatter_w" if with_weight else "sc_scatter",
    )


_sc_scatter_weighted = _make_sc_scatter(True)
_sc_scatter_plain = _make_sc_scatter(False)


def _mm(a, w):
    return lax.dot_general(a, w, (((1,), (1,)), ((), ())),
                           preferred_element_type=jnp.float32)


def _tc_stage_a(x_ref, wr_ref, wq_ref, br_ref, bq_ref, y_ref, r_ref):
    x = x_ref[...]
    y_ref[...] = _mm(x, wr_ref[...])
    r_ref[...] = _mm(x, wq_ref[...]) + br_ref[...] + bq_ref[...]


def _tc_stage_b(p_ref, r_ref, wr_ref, wq_ref, br_ref, bq_ref, y_ref, r2_ref):
    h = jnp.maximum(p_ref[0] + p_ref[1] + r_ref[...], 0.0)
    y_ref[...] = _mm(h, wr_ref[...])
    r2_ref[...] = _mm(h, wq_ref[...]) + br_ref[...] + bq_ref[...]


def _tc_stage_c(p_ref, r_ref, o_ref):
    o_ref[...] = jnp.maximum(p_ref[0] + p_ref[1] + r_ref[...], 0.0)


_nd = jax.ShapeDtypeStruct((N, D), jnp.float32)

_stage_a = pl.pallas_call(_tc_stage_a, out_shape=(_nd, _nd))
_stage_b = pl.pallas_call(_tc_stage_b, out_shape=(_nd, _nd))
_stage_c = pl.pallas_call(_tc_stage_c, out_shape=_nd)


def kernel(x, edge_index, edge_weight, W1_rel, b1_rel, W1_root, b1_root,
           W2_rel, b2_rel, W2_root, b2_root):
    src = edge_index[0].reshape(NW, NSB, CB, C)
    dst = edge_index[1].reshape(NW, NSB, CB, C)
    w = edge_weight.reshape(NW, NSB, CB * C)
    b1r = b1_rel.reshape(1, D)
    b1q = b1_root.reshape(1, D)
    b2r = b2_rel.reshape(1, D)
    b2q = b2_root.reshape(1, D)

    y1, r1 = _stage_a(x, W1_rel, W1_root, b1r, b1q)
    p1 = _sc_scatter_weighted(y1, src, dst, w)[:, :N, :]
    y2, r2 = _stage_b(p1, r1, W2_rel, W2_root, b2r, b2q)
    p2 = _sc_scatter_plain(y2, src, dst, w)[:, :N, :]
    return _stage_c(p2, r2)


# speedup vs baseline: 9.9263x; 1.5236x over previous
import functools

import jax
import jax.numpy as jnp
from jax import lax
from jax.experimental import pallas as pl
from jax.experimental.pallas import tpu as pltpu
from jax.experimental.pallas import tpu_sc as plsc

N = 10000
E = 320000
D = 128

NC = 2
NS = 16
NW = NC * NS
EW = E // NW
C = 80
NCH = EW // C
NP = 10240
RPT = NP // NS
CB = 25
NSB = NCH // CB
LANES = 16


def _sc_scatter_body(with_weight, y_hbm, src_hbm, dst_hbm, w_hbm, out_hbm,
                     src_v, dst_v, w_v, rows_a, rows_b, acc_sh, sem_a, sem_b):
    cid = lax.axis_index("c")
    sid = lax.axis_index("s")
    wid = sid * NC + cid
    rows = (rows_a, rows_b)
    sems = (sem_a, sem_b)

    def zero_body(i, _):
        rows_a[i // 8, pl.ds((i % 8) * LANES, LANES)] = jnp.zeros(
            (LANES,), jnp.float32)
        return 0
    lax.fori_loop(0, C * 8, zero_body, 0)
    for z in range(RPT // C):
        pltpu.sync_copy(rows_a, acc_sh.at[pl.ds(sid * RPT + z * C, C)])
    plsc.subcore_barrier()

    def start_gather(j, buf):
        pltpu.async_copy(y_hbm.at[src_v.at[j]], rows[buf], sems[buf])

    def wait_gather(j, buf):
        pltpu.make_async_copy(y_hbm.at[src_v.at[j]], rows[buf],
                              sems[buf]).wait()

    def scale_scatter(j, buf):
        r = rows[buf]
        if with_weight:
            def group_body(g, _):
                wv = w_v[pl.ds(j * C + g * LANES, LANES)]
                for el in range(LANES):
                    e = g * LANES + el
                    wb = jnp.full((LANES,), wv[el], jnp.float32)
                    for k in range(D // LANES):
                        sl = pl.ds(k * LANES, LANES)
                        r[e, sl] = r[e, sl] * wb
                return 0
            lax.fori_loop(0, C // LANES, group_body, 0)
        pltpu.sync_copy(r, acc_sh.at[dst_v.at[j]], add=True)

    def sblock_body(sb, _):
        pltpu.sync_copy(src_hbm.at[wid, sb], src_v)
        pltpu.sync_copy(dst_hbm.at[wid, sb], dst_v)
        if with_weight:
            pltpu.sync_copy(w_hbm.at[wid, sb], w_v)
        start_gather(0, 0)

        def pair_body(s, _):
            a = 2 * s
            wait_gather(a, 0)
            start_gather(a + 1, 1)
            scale_scatter(a, 0)
            start_gather(a + 2, 0)
            wait_gather(a + 1, 1)
            scale_scatter(a + 1, 1)
            return 0
        lax.fori_loop(0, (CB - 1) // 2, pair_body, 0)
        wait_gather(CB - 1, 0)
        scale_scatter(CB - 1, 0)
        return 0
    lax.fori_loop(0, NSB, sblock_body, 0)
    plsc.subcore_barrier()

    for z in range(RPT // C):
        r0 = sid * RPT + z * C
        pltpu.sync_copy(acc_sh.at[pl.ds(r0, C)], rows_a)
        pltpu.sync_copy(rows_a, out_hbm.at[cid, pl.ds(r0, C)])


def _make_sc_scatter(with_weight):
    mesh = plsc.VectorSubcoreMesh(core_axis_name="c", subcore_axis_name="s",
                                  num_cores=NC, num_subcores=NS)
    scratch = [
        pltpu.VMEM((CB, C), jnp.int32),
        pltpu.VMEM((CB, C), jnp.int32),
        pltpu.VMEM((CB * C,), jnp.float32),
        pltpu.VMEM((C, D), jnp.float32),
        pltpu.VMEM((C, D), jnp.float32),
        pltpu.VMEM_SHARED((NP, D), jnp.float32),
        pltpu.SemaphoreType.DMA,
        pltpu.SemaphoreType.DMA,
    ]
    return pl.kernel(
        functools.partial(_sc_scatter_body, with_weight),
        out_type=jax.ShapeDtypeStruct((NC, NP, D), jnp.float32),
        mesh=mesh,
        scratch_types=scratch,
        name="sc_scatter_w" if with_weight else "sc_scatter",
    )


_sc_scatter_weighted = _make_sc_scatter(True)
_sc_scatter_plain = _make_sc_scatter(False)


def _mm(a, w):
    return lax.dot_general(a, w, (((1,), (1,)), ((), ())),
                           preferred_element_type=jnp.float32)


def _tc_stage_a(x_ref, wr_ref, wq_ref, br_ref, bq_ref, y_ref, r_ref):
    x = x_ref[...]
    y_ref[...] = _mm(x, wr_ref[...])
    r_ref[...] = _mm(x, wq_ref[...]) + br_ref[...] + bq_ref[...]


def _tc_stage_b(p_ref, r_ref, wr_ref, wq_ref, br_ref, bq_ref, y_ref, r2_ref):
    h = jnp.maximum(p_ref[0] + p_ref[1] + r_ref[...], 0.0)
    y_ref[...] = _mm(h, wr_ref[...])
    r2_ref[...] = _mm(h, wq_ref[...]) + br_ref[...] + bq_ref[...]


def _tc_stage_c(p_ref, r_ref, o_ref):
    o_ref[...] = jnp.maximum(p_ref[0] + p_ref[1] + r_ref[...], 0.0)


_nd = jax.ShapeDtypeStruct((N, D), jnp.float32)

_stage_a = pl.pallas_call(_tc_stage_a, out_shape=(_nd, _nd))
_stage_b = pl.pallas_call(_tc_stage_b, out_shape=(_nd, _nd))
_stage_c = pl.pallas_call(_tc_stage_c, out_shape=_nd)


def kernel(x, edge_index, edge_weight, W1_rel, b1_rel, W1_root, b1_root,
           W2_rel, b2_rel, W2_root, b2_root):
    src = edge_index[0].reshape(NW, NSB, CB, C)
    dst = edge_index[1].reshape(NW, NSB, CB, C)
    w = edge_weight.reshape(NW, NSB, CB * C)
    b1r = b1_rel.reshape(1, D)
    b1q = b1_root.reshape(1, D)
    b2r = b2_rel.reshape(1, D)
    b2q = b2_root.reshape(1, D)

    y1, r1 = _stage_a(x, W1_rel, W1_root, b1r, b1q)
    p1 = _sc_scatter_weighted(y1, src, dst, w)[:, :N, :]
    y2, r2 = _stage_b(p1, r1, W2_rel, W2_root, b2r, b2q)
    p2 = _sc_scatter_plain(y2, src, dst, w)[:, :N, :]
    return _stage_c(p2, r2)


# speedup vs baseline: 11.9507x; 1.2039x over previous
import functools

import jax
import jax.numpy as jnp
from jax import lax
from jax.experimental import pallas as pl
from jax.experimental.pallas import tpu as pltpu
from jax.experimental.pallas import tpu_sc as plsc

N = 10000
E = 320000
D = 128

NC = 2
NS = 16
NW = NC * NS
EW = E // NW
C = 80
NCH = EW // C
NP = 10240
RPT = NP // NS
CB = 25
NSB = NCH // CB
LANES = 16


def _sc_scatter_body(with_weight, y_hbm, src_hbm, dst_hbm, w_hbm, out_hbm,
                     src_v, dst_v, w_v, rows_0, rows_1, rows_2, acc_sh,
                     gs_0, gs_1, gs_2, ss_0, ss_1, ss_2):
    cid = lax.axis_index("c")
    sid = lax.axis_index("s")
    wid = sid * NC + cid
    rows = (rows_0, rows_1, rows_2)
    gsem = (gs_0, gs_1, gs_2)
    ssem = (ss_0, ss_1, ss_2)

    def zero_body(i, _):
        rows_0[i // 8, pl.ds((i % 8) * LANES, LANES)] = jnp.zeros(
            (LANES,), jnp.float32)
        return 0
    lax.fori_loop(0, C * 8, zero_body, 0)
    for z in range(RPT // C):
        pltpu.sync_copy(rows_0, acc_sh.at[pl.ds(sid * RPT + z * C, C)])
    plsc.subcore_barrier()

    def start_gather(j, b):
        pltpu.async_copy(y_hbm.at[src_v.at[j]], rows[b], gsem[b])

    def wait_gather(j, b):
        pltpu.make_async_copy(y_hbm.at[src_v.at[j]], rows[b], gsem[b]).wait()

    def scale(j, b):
        r = rows[b]

        def group_body(g, _):
            wv = w_v[pl.ds(j * C + g * LANES, LANES)]
            for el in range(LANES):
                e = g * LANES + el
                wb = jnp.full((LANES,), wv[el], jnp.float32)
                for k in range(D // LANES):
                    sl = pl.ds(k * LANES, LANES)
                    r[e, sl] = r[e, sl] * wb
            return 0
        lax.fori_loop(0, C // LANES, group_body, 0)

    def start_scatter(j, b):
        pltpu.async_copy(rows[b], acc_sh.at[dst_v.at[j]], ssem[b], add=True)

    def wait_scatter(j, b):
        pltpu.make_async_copy(rows[b], acc_sh.at[dst_v.at[j]],
                              ssem[b]).wait()

    def lane(j, b, wait_prev, next_j):
        wait_gather(j, b)
        if with_weight:
            scale(j, b)
        start_scatter(j, b)
        if wait_prev:
            wait_scatter(j - 1, (b + 2) % 3)
        if next_j:
            start_gather(j + 2, (b + 2) % 3)

    def sblock_body(sb, _):
        pltpu.sync_copy(src_hbm.at[wid, sb], src_v)
        pltpu.sync_copy(dst_hbm.at[wid, sb], dst_v)
        if with_weight:
            pltpu.sync_copy(w_hbm.at[wid, sb], w_v)

        start_gather(0, 0)
        start_gather(1, 1)
        lane(0, 0, wait_prev=False, next_j=True)
        lane(1, 1, wait_prev=True, next_j=True)

        def ring_body(t, _):
            j0 = 2 + 3 * t
            lane(j0, 2, wait_prev=True, next_j=True)
            lane(j0 + 1, 0, wait_prev=True, next_j=True)
            lane(j0 + 2, 1, wait_prev=True, next_j=True)
            return 0
        lax.fori_loop(0, (CB - 4) // 3, ring_body, 0)

        lane(CB - 2, 2, wait_prev=True, next_j=False)
        lane(CB - 1, 0, wait_prev=True, next_j=False)
        wait_scatter(CB - 1, 0)
        return 0
    lax.fori_loop(0, NSB, sblock_body, 0)
    plsc.subcore_barrier()

    for z in range(RPT // C):
        r0 = sid * RPT + z * C
        pltpu.sync_copy(acc_sh.at[pl.ds(r0, C)], rows_0)
        pltpu.sync_copy(rows_0, out_hbm.at[cid, pl.ds(r0, C)])


def _make_sc_scatter(with_weight):
    mesh = plsc.VectorSubcoreMesh(core_axis_name="c", subcore_axis_name="s",
                                  num_cores=NC, num_subcores=NS)
    scratch = [
        pltpu.VMEM((CB, C), jnp.int32),
        pltpu.VMEM((CB, C), jnp.int32),
        pltpu.VMEM((CB * C,), jnp.float32),
        pltpu.VMEM((C, D), jnp.float32),
        pltpu.VMEM((C, D), jnp.float32),
        pltpu.VMEM((C, D), jnp.float32),
        pltpu.VMEM_SHARED((NP, D), jnp.float32),
        pltpu.SemaphoreType.DMA,
        pltpu.SemaphoreType.DMA,
        pltpu.SemaphoreType.DMA,
        pltpu.SemaphoreType.DMA,
        pltpu.SemaphoreType.DMA,
        pltpu.SemaphoreType.DMA,
    ]
    return pl.kernel(
        functools.partial(_sc_scatter_body, with_weight),
        out_type=jax.ShapeDtypeStruct((NC, NP, D), jnp.float32),
        mesh=mesh,
        scratch_types=scratch,
        name="sc_scatter_w" if with_weight else "sc_scatter",
    )


_sc_scatter_weighted = _make_sc_scatter(True)
_sc_scatter_plain = _make_sc_scatter(False)


def _mm(a, w):
    return lax.dot_general(a, w, (((1,), (1,)), ((), ())),
                           preferred_element_type=jnp.float32)


def _tc_layer(p_ref, x_ref, wr_ref, wq_ref, br_ref, bq_ref, h_ref):
    agg = p_ref[0, :N, :] + p_ref[1, :N, :]
    h = _mm(agg, wr_ref[...]) + _mm(x_ref[...], wq_ref[...])
    h_ref[...] = jnp.maximum(h + br_ref[...] + bq_ref[...], 0.0)


_nd = jax.ShapeDtypeStruct((N, D), jnp.float32)
_tc_layer_call = pl.pallas_call(_tc_layer, out_shape=_nd)


def kernel(x, edge_index, edge_weight, W1_rel, b1_rel, W1_root, b1_root,
           W2_rel, b2_rel, W2_root, b2_root):
    src = edge_index[0].reshape(NW, NSB, CB, C)
    dst = edge_index[1].reshape(NW, NSB, CB, C)
    w = edge_weight.reshape(NW, NSB, CB * C)

    p1 = _sc_scatter_weighted(x, src, dst, w)
    h1 = _tc_layer_call(p1, x, W1_rel, W1_root,
                        b1_rel.reshape(1, D), b1_root.reshape(1, D))
    p2 = _sc_scatter_plain(h1, src, dst, w)
    return _tc_layer_call(p2, h1, W2_rel, W2_root,
                          b2_rel.reshape(1, D), b2_root.reshape(1, D))
